# Initial kernel scaffold; baseline (speedup 1.0000x reference)
import jax
import jax.numpy as jnp
from jax.experimental import pallas as pl


def kernel(author_ids, topic_ids, auth_cnts, topic_cnts, paper_ids, id_maps, edge_indexs, paper_sets, author_sets, topic_sets, topic_embs, au_embs, paper_embs, W_topic, b_topic, W_au, b_au, W_paper, b_paper, W_gat, att_src, att_dst, b_gat, W_p1, b_p1, W_p2, b_p2, W_p3, b_p3):
    raise NotImplementedError("write your pallas kernel here")


# speedup vs baseline: 73.6604x; 73.6604x over previous
import functools

import jax
import jax.numpy as jnp
from jax import lax
from jax.experimental import pallas as pl
from jax.experimental.pallas import tpu as pltpu
from jax.experimental.pallas import tpu_sc as plsc

N = 10000
E = 320000
D = 128
ROWW = 256
EPT = E // 32
MCAP = EPT + 16


def _prep_body(topic_ref, au_ref, paper_ref, wt_ref, bt_ref, wa_ref, ba_ref,
               wp_ref, bp_ref, wg_ref, att_ref, cat2_ref, pq_ref):
    t = topic_ref[...] @ wt_ref[...] + bt_ref[...]
    a = au_ref[...] @ wa_ref[...] + ba_ref[...]
    p = paper_ref[...] @ wp_ref[...] + bp_ref[...]
    cat2 = jnp.concatenate([jnp.zeros((1, D), jnp.float32), t, a, p], axis=0)
    cat2_ref[...] = cat2
    uv = wg_ref[...] @ att_ref[...]
    pq_ref[...] = (cat2 @ uv).T


def _prep(topic, au, paper, wt, bt, wa, ba, wp, bp, wg, att):
    return pl.pallas_call(
        _prep_body,
        out_shape=(jax.ShapeDtypeStruct((N, D), jnp.float32),
                   jax.ShapeDtypeStruct((2, N), jnp.float32)),
    )(topic, au, paper, wt, bt, wa, ba, wp, bp, wg, att)


def _mk_tables():
    mesh = plsc.VectorSubcoreMesh(core_axis_name="c", subcore_axis_name="s")

    @functools.partial(
        pl.kernel,
        out_type=(jax.ShapeDtypeStruct((N,), jnp.int32),
                  jax.ShapeDtypeStruct((N,), jnp.int32),
                  jax.ShapeDtypeStruct((N,), jnp.float32),
                  jax.ShapeDtypeStruct((N,), jnp.float32)),
        mesh=mesh,
        compiler_params=pltpu.CompilerParams(needs_layout_passes=False),
        scratch_types=[
            pltpu.VMEM((N,), jnp.int32),
            pltpu.VMEM((N,), jnp.int32),
            pltpu.VMEM((N,), jnp.int32),
            pltpu.VMEM((N,), jnp.float32),
            pltpu.VMEM((N,), jnp.float32),
            pltpu.VMEM((N,), jnp.float32),
            pltpu.VMEM((N,), jnp.float32),
            pltpu.VMEM((16,), jnp.int32),
        ],
    )
    def tables_kernel(sets_hbm, pq_hbm, ids_hbm, win_hbm, tmap_hbm, asrc_hbm,
                      adst_hbm, sets_v, win_v, tmap_v, ptab_v, qtab_v, asrc_v,
                      adst_v, ids_v):
        c = lax.axis_index("c")
        s = lax.axis_index("s")
        iota = lax.iota(jnp.int32, 16)

        def dedup_keep_last(idx):
            keep = jnp.ones((16,), jnp.bool_)
            for k in range(1, 16):
                sh = jnp.take(idx, jnp.minimum(iota + k, 15), mode="wrap")
                keep = keep & ~((iota + k <= 15) & (sh == idx))
            return keep

        @pl.when((c == 0) & (s == 0))
        def _():
            pltpu.sync_copy(sets_hbm, sets_v)
            pltpu.sync_copy(pq_hbm.at[0], ptab_v)
            pltpu.sync_copy(pq_hbm.at[1], qtab_v)
            pltpu.sync_copy(ids_hbm, ids_v)

            def zero_body(i, _):
                win_v[pl.ds(i * 16, 16)] = jnp.zeros((16,), jnp.int32)
                tmap_v[pl.ds(i * 16, 16)] = jnp.full((16,), -1, jnp.int32)
                return 0

            lax.fori_loop(0, N // 16, zero_body, 0)

            def win_body(i, _):
                idx = sets_v[pl.ds(i * 16, 16)]
                keep = dedup_keep_last(idx)
                plsc.store_scatter(win_v, [idx], i * 16 + iota, mask=keep)
                return 0

            lax.fori_loop(0, N // 16, win_body, 0)

            ids = ids_v[...]
            keep = dedup_keep_last(ids)
            plsc.store_scatter(tmap_v, [ids], iota, mask=keep)

            def gath_body(i, _):
                wv = win_v[pl.ds(i * 16, 16)]
                asrc_v[pl.ds(i * 16, 16)] = plsc.load_gather(ptab_v, [wv])
                adst_v[pl.ds(i * 16, 16)] = plsc.load_gather(qtab_v, [wv])
                return 0

            lax.fori_loop(0, N // 16, gath_body, 0)

            pltpu.sync_copy(win_v, win_hbm)
            pltpu.sync_copy(tmap_v, tmap_hbm)
            pltpu.sync_copy(asrc_v, asrc_hbm)
            pltpu.sync_copy(adst_v, adst_hbm)

    return tables_kernel


def _mk_edges():
    mesh = plsc.VectorSubcoreMesh(core_axis_name="c", subcore_axis_name="s")

    @functools.partial(
        pl.kernel,
        out_type=jax.ShapeDtypeStruct((2, 16, ROWW), jnp.float32),
        mesh=mesh,
        compiler_params=pltpu.CompilerParams(needs_layout_passes=False),
        scratch_types=[
            pltpu.VMEM((N,), jnp.int32),
            pltpu.VMEM((N,), jnp.int32),
            pltpu.VMEM((N,), jnp.float32),
            pltpu.VMEM((N,), jnp.float32),
            pltpu.VMEM((EPT,), jnp.int32),
            pltpu.VMEM((EPT,), jnp.int32),
            pltpu.VMEM((MCAP,), jnp.int32),
            pltpu.VMEM((MCAP,), jnp.int32),
            pltpu.VMEM((MCAP,), jnp.float32),
            pltpu.VMEM((16, D), jnp.float32),
            pltpu.VMEM((16, ROWW), jnp.float32),
            pltpu.VMEM((16, ROWW), jnp.float32),
            pltpu.VMEM((16, ROWW), jnp.float32),
            pltpu.VMEM((16,), jnp.int32),
            pltpu.VMEM_SHARED((16, 16, ROWW), jnp.float32),
            pltpu.SemaphoreType.DMA,
        ],
    )
    def edges_kernel(src_hbm, dst_hbm, win_hbm, asrc_hbm, adst_hbm, tmap_hbm,
                     cat2_hbm, ids_hbm, part_hbm, tmap_v, win_v, asrc_v,
                     adst_v, src_v, dst_v, mrow_v, mslot_v, mw_v, rows_v,
                     gf_v, res_v, res2_v, ids_v, acc_sh, sem):
        c = lax.axis_index("c")
        s = lax.axis_index("s")
        wid = c * 16 + s
        iota = lax.iota(jnp.int32, 16)
        zf = jnp.zeros((16,), jnp.float32)

        pltpu.sync_copy(tmap_hbm, tmap_v)
        pltpu.sync_copy(win_hbm, win_v)
        pltpu.sync_copy(asrc_hbm, asrc_v)
        pltpu.sync_copy(adst_hbm, adst_v)
        pltpu.sync_copy(ids_hbm, ids_v)
        pltpu.sync_copy(src_hbm.at[pl.ds(wid * EPT, EPT)], src_v)
        pltpu.sync_copy(dst_hbm.at[pl.ds(wid * EPT, EPT)], dst_v)

        for r in range(16):
            for k in range(ROWW // 16):
                gf_v[r, pl.ds(k * 16, 16)] = zf

        def zm_body(i, _):
            mrow_v[pl.ds(i * 16, 16)] = jnp.zeros((16,), jnp.int32)
            mslot_v[pl.ds(i * 16, 16)] = jnp.zeros((16,), jnp.int32)
            mw_v[pl.ds(i * 16, 16)] = zf
            return 0

        lax.fori_loop(0, MCAP // 16, zm_body, 0)

        def emit(cnt, wr, sv, w, m):
            plsc.store_compressed(mrow_v.at[pl.ds(cnt, 16)], wr, mask=m)
            plsc.store_compressed(mslot_v.at[pl.ds(cnt, 16)], sv, mask=m)
            plsc.store_compressed(mw_v.at[pl.ds(cnt, 16)], w, mask=m)
            return cnt + jnp.sum(m.astype(jnp.int32))

        def weight(srcv, dstv):
            al = (plsc.load_gather(asrc_v, [srcv]) +
                  plsc.load_gather(adst_v, [dstv]))
            al = jnp.where(al >= 0, al, 0.2 * al)
            return jnp.exp(al)

        def p1_body(i, cnt):
            dstv = dst_v[pl.ds(i * 16, 16)]
            srcv = src_v[pl.ds(i * 16, 16)]
            sv = plsc.load_gather(tmap_v, [dstv])
            m = sv >= 0
            w = weight(srcv, dstv)
            wr = plsc.load_gather(win_v, [srcv])
            return emit(cnt, wr, sv, w, m)

        cnt = lax.fori_loop(0, EPT // 16, p1_body, jnp.int32(0))

        ids = ids_v[...]
        svl = plsc.load_gather(tmap_v, [ids])
        msl = (svl == iota) & (wid == 0)
        wsl = weight(ids, ids)
        wrl = plsc.load_gather(win_v, [ids])
        cnt = emit(cnt, wrl, svl, wsl, msl)

        nch = (cnt + 15) // 16

        def p2_body(mi, _):
            pltpu.async_copy(cat2_hbm.at[mrow_v.at[pl.ds(mi * 16, 16)]],
                             rows_v, sem).wait()
            msv = mslot_v[pl.ds(mi * 16, 16)]
            mwv = mw_v[pl.ds(mi * 16, 16)]
            dcol = jnp.where(iota == 0, 1.0, 0.0)
            for i in range(16):
                lane = jnp.full((16,), i, jnp.int32)
                svb = jnp.take(msv, lane, mode="wrap")
                wvb = jnp.take(mwv, lane, mode="wrap")
                for k in range(D // 16):
                    plsc.addupdate_scatter(
                        gf_v, [svb, k * 16 + iota],
                        wvb * rows_v[i, pl.ds(k * 16, 16)])
                plsc.addupdate_scatter(gf_v, [svb, 128 + iota], wvb * dcol)
            return 0

        lax.fori_loop(0, nch, p2_body, 0)

        pltpu.sync_copy(gf_v, acc_sh.at[s])
        plsc.subcore_barrier()

        @pl.when(s == 0)
        def _():
            def red_body(t, _):
                pltpu.sync_copy(acc_sh.at[t], res_v)
                for r in range(16):
                    for k in range(ROWW // 16):
                        gf_v[r, pl.ds(k * 16, 16)] = (
                            gf_v[r, pl.ds(k * 16, 16)] +
                            res_v[r, pl.ds(k * 16, 16)])
                return 0

            lax.fori_loop(1, 16, red_body, 0)
            slotsrc = plsc.load_gather(tmap_v, [ids_v[...]])
            for j in range(16):
                rj = jnp.take(slotsrc, jnp.full((16,), j, jnp.int32),
                              mode="wrap")
                for k in range(ROWW // 16):
                    res2_v[j, pl.ds(k * 16, 16)] = plsc.load_gather(
                        gf_v, [rj, k * 16 + iota])
            pltpu.sync_copy(res2_v, part_hbm.at[c])

    return edges_kernel


def _final_body(part_ref, wg_ref, bg_ref, ids_ref, cnt_ref, wp1_ref, bp1_ref,
                wp2_ref, bp2_ref, wp3_ref, bp3_ref, out_ref):
    acc = part_ref[0] + part_ref[1]
    g = acc[:, :D]
    den = acc[:, D:D + 1]
    nf = g @ wg_ref[...] / (den + 1e-16) + bg_ref[...]
    nf = jnp.maximum(nf, 0.0)
    nf = jnp.where((ids_ref[...] == 0).reshape(16, 1), jnp.float32(1e-5), nf)
    t = jnp.sum(nf[8:16], axis=0, keepdims=True) / (1e-5 + cnt_ref[0, 1])
    arest = jnp.sum(nf[1:8], axis=0, keepdims=True) / (-1.00001 + cnt_ref[0, 0])
    cat = jnp.concatenate([nf[0:1], arest, t], axis=1)
    o = jnp.tanh(cat @ wp1_ref[...] + bp1_ref[...])
    o = jnp.tanh(o @ wp2_ref[...] + bp2_ref[...])
    out_ref[...] = o @ wp3_ref[...] + bp3_ref[...]


def _final(part, wg, bg, ids, cnt, wp1, bp1, wp2, bp2, wp3, bp3):
    return pl.pallas_call(
        _final_body,
        out_shape=jax.ShapeDtypeStruct((1, 1), jnp.float32),
    )(part, wg, bg, ids, cnt, wp1, bp1, wp2, bp2, wp3, bp3)


_TABLES = _mk_tables()
_EDGES = _mk_edges()


def kernel(author_ids, topic_ids, auth_cnts, topic_cnts, paper_ids, id_maps,
           edge_indexs, paper_sets, author_sets, topic_sets, topic_embs,
           au_embs, paper_embs, W_topic, b_topic, W_au, b_au, W_paper,
           b_paper, W_gat, att_src, att_dst, b_gat, W_p1, b_p1, W_p2, b_p2,
           W_p3, b_p3):
    att = jnp.stack([att_src, att_dst], axis=1)
    cat2, pq = _prep(topic_embs[0], au_embs[0], paper_embs[0], W_topic,
                     b_topic.reshape(1, D), W_au, b_au.reshape(1, D),
                     W_paper, b_paper.reshape(1, D), W_gat, att)

    allsets = jnp.concatenate([jnp.zeros((1,), jnp.int32), topic_sets[0],
                               author_sets[0], paper_sets[0]])
    ids16 = jnp.concatenate([author_ids[0], topic_ids[0]])
    winner, tmap, asrc, adst = _TABLES(allsets, pq, ids16)

    part = _EDGES(edge_indexs[0, 0], edge_indexs[0, 1], winner, asrc, adst,
                  tmap, cat2, ids16)

    cnt = jnp.stack([auth_cnts[0], topic_cnts[0]]).astype(jnp.float32)
    out = _final(part, W_gat, b_gat.reshape(1, D), ids16.reshape(16, 1),
                 cnt.reshape(1, 2), W_p1, b_p1.reshape(1, 2 * D), W_p2,
                 b_p2.reshape(1, D), W_p3, b_p3.reshape(1, 1))
    return out.reshape(-1)


# speedup vs baseline: 97.2115x; 1.3197x over previous
import functools

import jax
import jax.numpy as jnp
from jax import lax
from jax.experimental import pallas as pl
from jax.experimental.pallas import tpu as pltpu
from jax.experimental.pallas import tpu_sc as plsc

N = 10000
E = 320000
D = 128
ROWW = 144
EPT = E // 32
MCAP = EPT + 16


def _prep_body(topic_ref, au_ref, paper_ref, ei_ref, wt_ref, bt_ref,
               wa_ref, ba_ref, wp_ref, bp_ref, wg_ref, att_ref,
               cat2_ref, pq_ref, e1_ref):
    t = topic_ref[...] @ wt_ref[...] + bt_ref[...]
    a = au_ref[...] @ wa_ref[...] + ba_ref[...]
    p = paper_ref[...] @ wp_ref[...] + bp_ref[...]
    cat2 = jnp.concatenate([jnp.zeros((1, D), jnp.float32), t, a, p], axis=0)
    cat2_ref[...] = cat2
    uv = wg_ref[...] @ att_ref[...]
    pq_ref[...] = (cat2 @ uv).T
    e1_ref[pl.ds(0, E)] = ei_ref[0]
    e1_ref[pl.ds(E, E)] = ei_ref[1]


def _prep(topic, au, paper, ei, wt, bt, wa, ba, wp, bp, wg, att):
    return pl.pallas_call(
        _prep_body,
        out_shape=(jax.ShapeDtypeStruct((N, D), jnp.float32),
                   jax.ShapeDtypeStruct((2, N), jnp.float32),
                   jax.ShapeDtypeStruct((2 * E,), jnp.int32)),
    )(topic, au, paper, ei, wt, bt, wa, ba, wp, bp, wg, att)


def _mk_tables():
    mesh = plsc.VectorSubcoreMesh(core_axis_name="c", subcore_axis_name="s")

    @functools.partial(
        pl.kernel,
        out_type=(jax.ShapeDtypeStruct((N,), jnp.int32),
                  jax.ShapeDtypeStruct((N,), jnp.int32),
                  jax.ShapeDtypeStruct((16,), jnp.int32)),
        mesh=mesh,
        compiler_params=pltpu.CompilerParams(needs_layout_passes=False),
        scratch_types=[
            pltpu.VMEM((N,), jnp.int32),
            pltpu.VMEM((N,), jnp.int32),
            pltpu.VMEM((N,), jnp.int32),
            pltpu.VMEM((16,), jnp.int32),
            pltpu.VMEM((16,), jnp.int32),
        ],
    )
    def tables_kernel(sets_hbm, ids_hbm, win_hbm, tmap_hbm, sl_hbm,
                      sets_v, win_v, tmap_v, ids_v, sl_v):
        c = lax.axis_index("c")
        s = lax.axis_index("s")
        iota = lax.iota(jnp.int32, 16)

        def scatter_keep_last(idx, val, ref):
            key = idx * 16 + iota
            sk, sval = plsc.sort_key_val(key, val)
            skn = jnp.take(sk, jnp.minimum(iota + 1, 15), mode="wrap")
            keep = (iota == 15) | ((sk >> 4) != (skn >> 4))
            plsc.store_scatter(ref, [sk >> 4], sval, mask=keep)

        @pl.when((c == 0) & (s == 0))
        def _():
            pltpu.sync_copy(sets_hbm, sets_v)
            pltpu.sync_copy(ids_hbm, ids_v)

            def zero_body(i, _):
                win_v[pl.ds(i * 16, 16)] = jnp.zeros((16,), jnp.int32)
                tmap_v[pl.ds(i * 16, 16)] = jnp.full((16,), -1, jnp.int32)
                return 0

            lax.fori_loop(0, N // 16, zero_body, 0)

            def win_body(i, _):
                idx = sets_v[pl.ds(i * 16, 16)]
                scatter_keep_last(idx, i * 16 + iota, win_v)
                return 0

            lax.fori_loop(0, N // 16, win_body, 0)

            scatter_keep_last(ids_v[...], iota, tmap_v)
            sl_v[...] = plsc.load_gather(tmap_v, [ids_v[...]])

            pltpu.sync_copy(win_v, win_hbm)
            pltpu.sync_copy(tmap_v, tmap_hbm)
            pltpu.sync_copy(sl_v, sl_hbm)

    return tables_kernel


def _mk_edges():
    mesh = plsc.VectorSubcoreMesh(core_axis_name="c", subcore_axis_name="s")

    @functools.partial(
        pl.kernel,
        out_type=jax.ShapeDtypeStruct((2, 16, 16, ROWW), jnp.float32),
        mesh=mesh,
        compiler_params=pltpu.CompilerParams(needs_layout_passes=False),
        scratch_types=[
            pltpu.VMEM((N,), jnp.int32),
            pltpu.VMEM((N,), jnp.int32),
            pltpu.VMEM((N,), jnp.float32),
            pltpu.VMEM((N,), jnp.float32),
            pltpu.VMEM((EPT,), jnp.int32),
            pltpu.VMEM((EPT,), jnp.int32),
            pltpu.VMEM((MCAP,), jnp.int32),
            pltpu.VMEM((MCAP,), jnp.int32),
            pltpu.VMEM((16, D), jnp.float32),
            pltpu.VMEM((16,), jnp.int32),
            pltpu.VMEM((16,), jnp.int32),
            pltpu.VMEM((16, ROWW), jnp.float32),
            pltpu.SemaphoreType.DMA,
        ],
    )
    def edges_kernel(ei_hbm, win_hbm, tmap_hbm, pq_hbm, cat2_hbm,
                     ids_hbm, part_hbm, tmap_v, win_v, ptab_v, qtab_v,
                     src_v, dst_v, mrow_v, mslot_v, rows_v, wrk_v, ids_v,
                     gf_v, sem):
        c = lax.axis_index("c")
        s = lax.axis_index("s")
        wid = c * 16 + s
        iota = lax.iota(jnp.int32, 16)
        zf = jnp.zeros((16,), jnp.float32)

        pltpu.sync_copy(tmap_hbm, tmap_v)
        pltpu.sync_copy(win_hbm, win_v)
        pltpu.sync_copy(pq_hbm.at[0], ptab_v)
        pltpu.sync_copy(pq_hbm.at[1], qtab_v)
        pltpu.sync_copy(ids_hbm, ids_v)
        pltpu.sync_copy(ei_hbm.at[pl.ds(wid * EPT, EPT)], src_v)
        pltpu.sync_copy(ei_hbm.at[pl.ds(E + wid * EPT, EPT)], dst_v)
        ids = ids_v[...]
        adt16 = plsc.load_gather(qtab_v, [plsc.load_gather(win_v, [ids])])

        for r in range(16):
            for k in range(ROWW // 16):
                gf_v[r, pl.ds(k * 16, 16)] = zf

        def zm_body(i, _):
            mrow_v[pl.ds(i * 16, 16)] = jnp.zeros((16,), jnp.int32)
            mslot_v[pl.ds(i * 16, 16)] = jnp.zeros((16,), jnp.int32)
            return 0

        lax.fori_loop(0, MCAP // 16, zm_body, 0)

        def emit(cnt, srcv, sv, m):
            plsc.store_compressed(mrow_v.at[pl.ds(cnt, 16)], srcv, mask=m)
            plsc.store_compressed(mslot_v.at[pl.ds(cnt, 16)], sv, mask=m)
            return cnt + jnp.sum(m.astype(jnp.int32))

        def p1_body(i, cnt):
            dstv = dst_v[pl.ds(i * 16, 16)]
            srcv = src_v[pl.ds(i * 16, 16)]
            sv = plsc.load_gather(tmap_v, [dstv])
            wr = plsc.load_gather(win_v, [srcv])
            return emit(cnt, wr, sv, sv >= 0)

        cnt = lax.fori_loop(0, EPT // 16, p1_body, jnp.int32(0))

        svl = plsc.load_gather(tmap_v, [ids])
        wrl = plsc.load_gather(win_v, [ids])
        cnt = emit(cnt, wrl, svl, (svl == iota) & (wid == 0))

        nch = (cnt + 15) // 16

        def p2_body(mi, _):
            wr = mrow_v[pl.ds(mi * 16, 16)]
            cps = []
            for i in range(16):
                ri = jnp.sum(jnp.where(iota == i, wr, 0))
                cps.append(
                    pltpu.async_copy(cat2_hbm.at[ri], rows_v.at[i], sem))
            valid = (mi * 16 + iota) < cnt
            svs = mslot_v[pl.ds(mi * 16, 16)]
            svc = jnp.minimum(jnp.maximum(svs, 0), 15)
            al = plsc.load_gather(ptab_v, [wr]) + jnp.take(
                adt16, svc, mode="wrap")
            al = jnp.where(al >= 0, al, 0.2 * al)
            w = jnp.where(valid, jnp.exp(al), 0.0)
            for cp in cps:
                cp.wait()
            dcol = jnp.where(iota == 0, 1.0, 0.0)
            for i in range(16):
                lane = jnp.full((16,), i, jnp.int32)
                svb = jnp.take(svc, lane, mode="wrap")
                wvb = jnp.take(w, lane, mode="wrap")
                for k in range(D // 16):
                    plsc.addupdate_scatter(
                        gf_v, [svb, k * 16 + iota],
                        wvb * rows_v[i, pl.ds(k * 16, 16)])
                plsc.addupdate_scatter(gf_v, [svb, 128 + iota], wvb * dcol)
            return 0

        lax.fori_loop(0, nch, p2_body, 0)

        pltpu.sync_copy(gf_v, part_hbm.at[c, s])

    return edges_kernel


def _final_body(part_ref, sl_ref, wg_ref, bg_ref, ids_ref, cnt_ref, wp1_ref,
                bp1_ref, wp2_ref, bp2_ref, wp3_ref, bp3_ref, out_ref):
    acc = jnp.sum(part_ref[...].reshape(32, 16, ROWW), axis=0)
    oh = (sl_ref[...].reshape(16, 1) ==
          lax.broadcasted_iota(jnp.int32, (1, 16), 1)).astype(jnp.float32)
    acc = oh @ acc
    g = acc[:, :D]
    den = acc[:, D:D + 1]
    nf = g @ wg_ref[...] / (den + 1e-16) + bg_ref[...]
    nf = jnp.maximum(nf, 0.0)
    nf = jnp.where((ids_ref[...] == 0).reshape(16, 1), jnp.float32(1e-5), nf)
    t = jnp.sum(nf[8:16], axis=0, keepdims=True) / (1e-5 + cnt_ref[0, 1])
    arest = jnp.sum(nf[1:8], axis=0, keepdims=True) / (-1.00001 + cnt_ref[0, 0])
    cat = jnp.concatenate([nf[0:1], arest, t], axis=1)
    o = jnp.tanh(cat @ wp1_ref[...] + bp1_ref[...])
    o = jnp.tanh(o @ wp2_ref[...] + bp2_ref[...])
    out_ref[...] = o @ wp3_ref[...] + bp3_ref[...]


def _final(part, sl, wg, bg, ids, cnt, wp1, bp1, wp2, bp2, wp3, bp3):
    return pl.pallas_call(
        _final_body,
        out_shape=jax.ShapeDtypeStruct((1, 1), jnp.float32),
    )(part, sl, wg, bg, ids, cnt, wp1, bp1, wp2, bp2, wp3, bp3)


_TABLES = _mk_tables()
_EDGES = _mk_edges()


def kernel(author_ids, topic_ids, auth_cnts, topic_cnts, paper_ids, id_maps,
           edge_indexs, paper_sets, author_sets, topic_sets, topic_embs,
           au_embs, paper_embs, W_topic, b_topic, W_au, b_au, W_paper,
           b_paper, W_gat, att_src, att_dst, b_gat, W_p1, b_p1, W_p2, b_p2,
           W_p3, b_p3):
    att = jnp.stack([att_src, att_dst], axis=1)
    allsets = jnp.concatenate([jnp.zeros((1,), jnp.int32), topic_sets[0],
                               author_sets[0], paper_sets[0]])
    ids16 = jnp.concatenate([author_ids[0], topic_ids[0]])
    cat2, pq, e1 = _prep(topic_embs[0], au_embs[0], paper_embs[0],
                         edge_indexs[0], W_topic, b_topic.reshape(1, D),
                         W_au, b_au.reshape(1, D), W_paper,
                         b_paper.reshape(1, D), W_gat, att)

    winner, tmap, slotsrc = _TABLES(allsets, ids16)

    part = _EDGES(e1, winner, tmap, pq, cat2, ids16)

    cnt = jnp.stack([auth_cnts[0], topic_cnts[0]]).astype(jnp.float32)
    out = _final(part, slotsrc.reshape(1, 16), W_gat,
                 b_gat.reshape(1, D), ids16.reshape(16, 1),
                 cnt.reshape(1, 2), W_p1, b_p1.reshape(1, 2 * D), W_p2,
                 b_p2.reshape(1, D), W_p3, b_p3.reshape(1, 1))
    return out.reshape(-1)
